# baseline (speedup 1.0000x reference)
import functools

import jax
import jax.numpy as jnp
from jax import lax
from jax.experimental import pallas as pl
from jax.experimental.pallas import tpu as pltpu
from jax.experimental.pallas import tpu_sc as plsc

SEQ = 8192
BATCH = 4
D_MODEL = 1024
LANES = 16

NUM_CORES = 2
NUM_SUBCORES = 16
NW = NUM_CORES * NUM_SUBCORES

CH = 4
CH4 = CH * BATCH
NBUF = 4
POS_PER_W = SEQ // NW
NCHUNK = POS_PER_W // CH
ROWS_PER_W = POS_PER_W * BATCH
ROUNDS = NCHUNK // NBUF
PREFETCH = 2


def _body(x_hbm, idx_hbm, tab_hbm, out_hbm, xbuf, pebuf, idxbuf,
          lsem, psem, ssem):
    cid = lax.axis_index("c")
    sid = lax.axis_index("s")
    wid = sid * NUM_CORES + cid
    row0 = wid * ROWS_PER_W

    pltpu.sync_copy(idx_hbm.at[wid], idxbuf)

    def issue_loads(j, t):
        pltpu.async_copy(
            x_hbm.at[pl.ds(row0 + t * CH4, CH4)], xbuf.at[j], lsem.at[j])
        pltpu.async_copy(tab_hbm.at[idxbuf.at[t]], pebuf.at[j], psem.at[j])

    def wait_loads(j, t):
        pltpu.make_async_copy(
            x_hbm.at[pl.ds(row0 + t * CH4, CH4)], xbuf.at[j],
            lsem.at[j]).wait()
        pltpu.make_async_copy(
            tab_hbm.at[pl.ds(0, CH)], pebuf.at[j], psem.at[j]).wait()

    def issue_store(j, t):
        pltpu.async_copy(
            xbuf.at[j], out_hbm.at[pl.ds(row0 + t * CH4, CH4)], ssem.at[j])

    def wait_store(j):
        pltpu.make_async_copy(
            xbuf.at[j], out_hbm.at[pl.ds(row0, CH4)], ssem.at[j]).wait()

    def compute(j):
        def p_body(p, _):
            def u_body(u, __):
                base = u * (LANES * 16)
                for v in range(16):
                    off = base + v * LANES
                    pe_v = pebuf[j, p, pl.ds(off, LANES)]
                    for b in range(BATCH):
                        plsc.addupdate(
                            xbuf.at[j, p * BATCH + b, pl.ds(off, LANES)],
                            pe_v)
                return 0
            lax.fori_loop(0, D_MODEL // (LANES * 16), u_body, 0)
            return 0
        lax.fori_loop(0, CH, p_body, 0)

    for t in range(PREFETCH):
        issue_loads(t % NBUF, t)

    @pl.loop(0, ROUNDS)
    def _round(r):
        for j in range(NBUF):
            t = r * NBUF + j
            wait_loads(j, t)
            compute(j)
            issue_store(j, t)
            nt = t + PREFETCH
            jn = (j + PREFETCH) % NBUF

            @pl.when(nt < NCHUNK)
            def _():
                @pl.when(nt >= NBUF)
                def _():
                    wait_store(jn)
                issue_loads(jn, nt)

    for j in range(NBUF):
        wait_store(j)


@jax.jit
def kernel(x, indices, lpe_weight):
    x2d = x.reshape(SEQ * BATCH, D_MODEL)
    idx3 = indices.reshape(NW, NCHUNK, CH)
    mesh = plsc.VectorSubcoreMesh(
        core_axis_name="c", subcore_axis_name="s",
        num_cores=NUM_CORES, num_subcores=NUM_SUBCORES)
    out2d = pl.kernel(
        _body,
        out_type=jax.ShapeDtypeStruct((SEQ * BATCH, D_MODEL), jnp.float32),
        mesh=mesh,
        scratch_types=[
            pltpu.VMEM((NBUF, CH4, D_MODEL), jnp.float32),
            pltpu.VMEM((NBUF, CH, D_MODEL), jnp.float32),
            pltpu.VMEM((NCHUNK, CH), jnp.int32),
            pltpu.SemaphoreType.DMA((NBUF,)),
            pltpu.SemaphoreType.DMA((NBUF,)),
            pltpu.SemaphoreType.DMA((NBUF,)),
        ],
    )(x2d, idx3, lpe_weight)
    return out2d.reshape(SEQ, BATCH, D_MODEL)
